# Initial kernel scaffold; baseline (speedup 1.0000x reference)
import jax
import jax.numpy as jnp
from jax.experimental import pallas as pl


def kernel(x, edge_index, edge_attr, A, Wg, bg, W1, b1, W2, b2):
    raise NotImplementedError("write your pallas kernel here")


# speedup vs baseline: 13.2531x; 13.2531x over previous
import functools

import jax
import jax.numpy as jnp
from jax import lax
from jax.experimental import pallas as pl
from jax.experimental.pallas import tpu as pltpu
from jax.experimental.pallas import tpu_sc as plsc

NC = 2
NS = 16
NW = NC * NS
CH = 128

def _sc_mesh():
  return plsc.VectorSubcoreMesh(
      core_axis_name="c", subcore_axis_name="s", num_cores=NC, num_subcores=NS
  )


def _make_deg_kernel(npad, nchunk, tslice):
  @functools.partial(
      pl.kernel,
      out_type=jax.ShapeDtypeStruct((NC * npad,), jnp.float32),
      mesh=_sc_mesh(),
      scratch_types=[
          pltpu.VMEM_SHARED((npad,), jnp.float32),
          pltpu.VMEM((nchunk, CH), jnp.int32),
          pltpu.VMEM((nchunk, CH), jnp.float32),
          pltpu.VMEM((tslice,), jnp.float32),
      ],
  )
  def deg_kernel(dst_hbm, w_hbm, out_hbm, deg_sh, didx, wv, zb):
    c = lax.axis_index("c")
    t = lax.axis_index("s")
    wid = c * NS + t

    def zf(k, _):
      zb[pl.ds(k * 16, 16)] = jnp.zeros((16,), jnp.float32)
      return 0

    lax.fori_loop(0, tslice // 16, zf, 0)
    pltpu.sync_copy(zb, deg_sh.at[pl.ds(t * tslice, tslice)])
    pltpu.sync_copy(dst_hbm.at[wid], didx)
    pltpu.sync_copy(w_hbm.at[wid], wv)
    plsc.subcore_barrier()

    def body(i, _):
      pltpu.sync_copy(wv.at[i], deg_sh.at[didx.at[i]], add=True)
      return 0

    lax.fori_loop(0, nchunk, body, 0)
    plsc.subcore_barrier()
    pltpu.sync_copy(deg_sh.at[pl.ds(t * tslice, tslice)], zb)
    pltpu.sync_copy(zb, out_hbm.at[pl.ds(c * npad + t * tslice, tslice)])

  return deg_kernel


def _make_agg_kernel(npad, nchunk, tslice, d):
  @functools.partial(
      pl.kernel,
      out_type=jax.ShapeDtypeStruct((NC * npad, d), jnp.float32),
      mesh=_sc_mesh(),
      scratch_types=[
          pltpu.VMEM_SHARED((npad, d), jnp.float32),
          pltpu.VMEM((CH,), jnp.int32),
          pltpu.VMEM((1, CH), jnp.int32),
          pltpu.VMEM((CH, 16), jnp.float32),
          pltpu.VMEM((CH, d), jnp.float32),
          pltpu.SemaphoreType.DMA,
      ],
  )
  def agg_kernel(src_hbm, dst_hbm, wrep_hbm, y_hbm, zeros_hbm, out_hbm,
                 acc_sh, sidx, didx, wrv, rows, gsem):
    c = lax.axis_index("c")
    t = lax.axis_index("s")
    wid = c * NS + t

    pltpu.sync_copy(zeros_hbm, rows)

    def zrb(j, _):
      pltpu.sync_copy(rows, acc_sh.at[pl.ds(t * tslice + j * CH, CH)])
      return 0

    lax.fori_loop(0, tslice // CH, zrb, 0)
    plsc.subcore_barrier()

    def body(i, _):
      pltpu.sync_copy(src_hbm.at[wid, i], sidx)
      gather = pltpu.async_copy(y_hbm.at[sidx], rows, gsem)
      pltpu.sync_copy(dst_hbm.at[wid, pl.ds(i, 1)], didx)
      pltpu.sync_copy(wrep_hbm.at[wid, i], wrv)
      gather.wait()

      def se(e, _):
        s = wrv[e, :]
        for j in range(d // 16):
          rows[e, pl.ds(j * 16, 16)] = rows[e, pl.ds(j * 16, 16)] * s
        return 0

      lax.fori_loop(0, CH, se, 0)
      pltpu.sync_copy(rows, acc_sh.at[didx.at[0]], add=True)
      return 0

    lax.fori_loop(0, nchunk, body, 0)
    plsc.subcore_barrier()
    def outb(j, _):
      base = t * tslice + j * CH
      pltpu.sync_copy(acc_sh.at[pl.ds(base, CH)], rows)
      pltpu.sync_copy(rows, out_hbm.at[pl.ds(c * npad + base, CH)])
      return 0

    lax.fori_loop(0, tslice // CH, outb, 0)

  return agg_kernel


def _y_body(pdegT_ref, x_ref, y_ref):
  pd = pdegT_ref[...]
  deg = 1.0 + pd[:, 0:1] + pd[:, 1:2]
  dinv = jnp.where(deg > 0, lax.rsqrt(deg), 0.0)
  y_ref[...] = x_ref[...] * dinv


def _mlp_body(pdegT_ref, acc_ref, y_ref, wg_ref, bg_ref, w1_ref, b1_ref,
              w2_ref, b2_ref, out_ref):
  pd = pdegT_ref[...]
  deg = 1.0 + pd[:, 0:1] + pd[:, 1:2]
  dinv = jnp.where(deg > 0, lax.rsqrt(deg), 0.0)
  acc = acc_ref[...]
  agg = (acc[0] + acc[1] + y_ref[...]) * dinv
  dn = (((1,), (1,)), ((), ()))
  h = lax.dot_general(agg, wg_ref[...], dn,
                      preferred_element_type=jnp.float32) + bg_ref[...]
  h = h * (1.0 / (1.0 + jnp.exp(-h)))
  z = lax.dot_general(h, w1_ref[...], dn,
                      preferred_element_type=jnp.float32) + b1_ref[...]
  z = jnp.where(z > 0, z, jnp.exp(z) - 1.0)
  hl = lax.dot_general(z, w2_ref[...], dn,
                       preferred_element_type=jnp.float32) + b2_ref[...]
  m = jnp.max(hl, axis=1, keepdims=True)
  ex = jnp.exp(hl - m)
  out_ref[...] = ex / jnp.sum(ex, axis=1, keepdims=True)


def kernel(x, edge_index, edge_attr, A, Wg, bg, W1, b1, W2, b2):
  n, d = x.shape
  e = edge_index.shape[1]
  h = Wg.shape[0]
  mlp = W1.shape[0]
  k = W2.shape[0]

  tslice = -(-n // (NS * CH)) * CH
  npad = tslice * NS
  epw = -(-e // (NW * CH)) * CH
  nchunk = epw // CH
  epad = epw * NW

  src = edge_index[0]
  dst = edge_index[1]
  ne_pad = epad - e
  pad_idx = jnp.arange(ne_pad, dtype=jnp.int32) % n
  src_p = jnp.concatenate([src, pad_idx])
  dst_p = jnp.concatenate([dst, pad_idx])
  w_p = jnp.concatenate([edge_attr, jnp.zeros((ne_pad,), jnp.float32)])
  src3 = src_p.reshape(NW, nchunk, CH)
  dst3 = dst_p.reshape(NW, nchunk, CH)
  w3 = w_p.reshape(NW, nchunk, CH)
  wrep = jnp.broadcast_to(w_p[:, None], (epad, 16)).reshape(NW, nchunk, CH, 16)
  zeros_rows = jnp.zeros((CH, d), jnp.float32)

  pdeg = _make_deg_kernel(npad, nchunk, tslice)(dst3, w3)
  pdegT = pdeg.reshape(NC, npad).T

  rb = 1000
  nb = n // rb
  y = pl.pallas_call(
      _y_body,
      grid=(nb,),
      in_specs=[
          pl.BlockSpec((rb, NC), lambda i: (i, 0)),
          pl.BlockSpec((rb, d), lambda i: (i, 0)),
      ],
      out_specs=pl.BlockSpec((rb, d), lambda i: (i, 0)),
      out_shape=jax.ShapeDtypeStruct((n, d), jnp.float32),
  )(pdegT, x)

  accs = _make_agg_kernel(npad, nchunk, tslice, d)(
      src3, dst3, wrep, y, zeros_rows).reshape(NC, npad, d)

  s = pl.pallas_call(
      _mlp_body,
      grid=(nb,),
      in_specs=[
          pl.BlockSpec((rb, NC), lambda i: (i, 0)),
          pl.BlockSpec((NC, rb, d), lambda i: (0, i, 0)),
          pl.BlockSpec((rb, d), lambda i: (i, 0)),
          pl.BlockSpec((h, d), lambda i: (0, 0)),
          pl.BlockSpec((1, h), lambda i: (0, 0)),
          pl.BlockSpec((mlp, h), lambda i: (0, 0)),
          pl.BlockSpec((1, mlp), lambda i: (0, 0)),
          pl.BlockSpec((k, mlp), lambda i: (0, 0)),
          pl.BlockSpec((1, k), lambda i: (0, 0)),
      ],
      out_specs=pl.BlockSpec((rb, k), lambda i: (i, 0)),
      out_shape=jax.ShapeDtypeStruct((n, k), jnp.float32),
  )(pdegT, accs, y, Wg, bg.reshape(1, h), W1, b1.reshape(1, mlp),
    W2, b2.reshape(1, k))

  return (A, s)


# speedup vs baseline: 13.2756x; 1.0017x over previous
import functools

import jax
import jax.numpy as jnp
from jax import lax
from jax.experimental import pallas as pl
from jax.experimental.pallas import tpu as pltpu
from jax.experimental.pallas import tpu_sc as plsc

NC = 2
NS = 16
NW = NC * NS
CH = 128

def _sc_mesh():
  return plsc.VectorSubcoreMesh(
      core_axis_name="c", subcore_axis_name="s", num_cores=NC, num_subcores=NS
  )


def _make_deg_kernel(npad, nchunk, tslice):
  @functools.partial(
      pl.kernel,
      out_type=jax.ShapeDtypeStruct((NC * npad,), jnp.float32),
      mesh=_sc_mesh(),
      scratch_types=[
          pltpu.VMEM_SHARED((npad,), jnp.float32),
          pltpu.VMEM((nchunk, CH), jnp.int32),
          pltpu.VMEM((nchunk, CH), jnp.float32),
          pltpu.VMEM((tslice,), jnp.float32),
      ],
  )
  def deg_kernel(dst_hbm, w_hbm, out_hbm, deg_sh, didx, wv, zb):
    c = lax.axis_index("c")
    t = lax.axis_index("s")
    wid = c * NS + t

    def zf(k, _):
      zb[pl.ds(k * 16, 16)] = jnp.zeros((16,), jnp.float32)
      return 0

    lax.fori_loop(0, tslice // 16, zf, 0)
    pltpu.sync_copy(zb, deg_sh.at[pl.ds(t * tslice, tslice)])
    pltpu.sync_copy(dst_hbm.at[wid], didx)
    pltpu.sync_copy(w_hbm.at[wid], wv)
    plsc.subcore_barrier()

    def body(i, _):
      pltpu.sync_copy(wv.at[i], deg_sh.at[didx.at[i]], add=True)
      return 0

    lax.fori_loop(0, nchunk, body, 0)
    plsc.subcore_barrier()
    pltpu.sync_copy(deg_sh.at[pl.ds(t * tslice, tslice)], zb)
    pltpu.sync_copy(zb, out_hbm.at[pl.ds(c * npad + t * tslice, tslice)])

  return deg_kernel


def _make_agg_kernel(npad, nchunk, tslice, d):
  @functools.partial(
      pl.kernel,
      out_type=jax.ShapeDtypeStruct((NC * npad, d), jnp.float32),
      mesh=_sc_mesh(),
      scratch_types=[
          pltpu.VMEM_SHARED((npad, d), jnp.float32),
          pltpu.VMEM((CH,), jnp.int32),
          pltpu.VMEM((1, CH), jnp.int32),
          pltpu.VMEM((CH, 16), jnp.float32),
          pltpu.VMEM((CH, d), jnp.float32),
          pltpu.SemaphoreType.DMA,
      ],
  )
  def agg_kernel(src_hbm, dst_hbm, w_hbm, y_hbm, zeros_hbm, out_hbm,
                 acc_sh, sidx, didx, wrv, rows, gsem):
    c = lax.axis_index("c")
    t = lax.axis_index("s")
    wid = c * NS + t

    pltpu.sync_copy(zeros_hbm, rows)
    nfull = tslice // CH
    rem = tslice % CH

    def zrb(j, _):
      pltpu.sync_copy(rows, acc_sh.at[pl.ds(t * tslice + j * CH, CH)])
      return 0

    lax.fori_loop(0, nfull, zrb, 0)
    if rem:
      pltpu.sync_copy(rows.at[pl.ds(0, rem)],
                      acc_sh.at[pl.ds(t * tslice + nfull * CH, rem)])
    plsc.subcore_barrier()

    def scale(rows_ref, wref):
      def se(e, _):
        s = wref[e, :]
        for j in range(d // 16):
          rows_ref[e, pl.ds(j * 16, 16)] = (
              rows_ref[e, pl.ds(j * 16, 16)] * s)
        return 0

      lax.fori_loop(0, CH, se, 0)

    def body(i, _):
      pltpu.sync_copy(src_hbm.at[wid, i], sidx)
      pltpu.async_copy(y_hbm.at[sidx], rows, gsem)
      pltpu.sync_copy(dst_hbm.at[wid, pl.ds(i, 1)], didx)
      pltpu.sync_copy(w_hbm.at[wid, i], wrv)
      pltpu.make_async_copy(y_hbm.at[sidx], rows, gsem).wait()
      scale(rows, wrv)
      pltpu.sync_copy(rows, acc_sh.at[didx.at[0]], add=True)
      return 0

    lax.fori_loop(0, nchunk, body, 0)
    plsc.subcore_barrier()

    def outb(j, _):
      base = t * tslice + j * CH
      pltpu.sync_copy(acc_sh.at[pl.ds(base, CH)], rows)
      pltpu.sync_copy(rows, out_hbm.at[pl.ds(c * npad + base, CH)])
      return 0

    lax.fori_loop(0, nfull, outb, 0)
    if rem:
      base = t * tslice + nfull * CH
      pltpu.sync_copy(acc_sh.at[pl.ds(base, rem)], rows.at[pl.ds(0, rem)])
      pltpu.sync_copy(rows.at[pl.ds(0, rem)],
                      out_hbm.at[pl.ds(c * npad + base, rem)])

  return agg_kernel


def _y_body(pdegT_ref, x_ref, y_ref):
  pd = pdegT_ref[...]
  deg = 1.0 + pd[:, 0:1] + pd[:, 1:2]
  dinv = jnp.where(deg > 0, lax.rsqrt(deg), 0.0)
  y_ref[...] = x_ref[...] * dinv


def _mlp_body(pdegT_ref, acc_ref, y_ref, wg_ref, bg_ref, w1_ref, b1_ref,
              w2_ref, b2_ref, out_ref):
  pd = pdegT_ref[...]
  deg = 1.0 + pd[:, 0:1] + pd[:, 1:2]
  dinv = jnp.where(deg > 0, lax.rsqrt(deg), 0.0)
  acc = acc_ref[...]
  agg = (acc[0] + acc[1] + y_ref[...]) * dinv
  dn = (((1,), (1,)), ((), ()))
  h = lax.dot_general(agg, wg_ref[...], dn,
                      preferred_element_type=jnp.float32) + bg_ref[...]
  h = h * (1.0 / (1.0 + jnp.exp(-h)))
  z = lax.dot_general(h, w1_ref[...], dn,
                      preferred_element_type=jnp.float32) + b1_ref[...]
  z = jnp.where(z > 0, z, jnp.exp(z) - 1.0)
  hl = lax.dot_general(z, w2_ref[...], dn,
                       preferred_element_type=jnp.float32) + b2_ref[...]
  m = jnp.max(hl, axis=1, keepdims=True)
  ex = jnp.exp(hl - m)
  out_ref[...] = ex / jnp.sum(ex, axis=1, keepdims=True)


def kernel(x, edge_index, edge_attr, A, Wg, bg, W1, b1, W2, b2):
  n, d = x.shape
  e = edge_index.shape[1]
  h = Wg.shape[0]
  mlp = W1.shape[0]
  k = W2.shape[0]

  tslice = -(-n // (NS * 8)) * 8
  npad = tslice * NS
  epw = -(-e // (NW * CH)) * CH
  nchunk = epw // CH
  epad = epw * NW

  src = edge_index[0]
  dst = edge_index[1]
  ne_pad = epad - e
  pad_idx = jnp.arange(ne_pad, dtype=jnp.int32) % n
  src_p = jnp.concatenate([src, pad_idx])
  dst_p = jnp.concatenate([dst, pad_idx])
  w_p = jnp.concatenate([edge_attr, jnp.zeros((ne_pad,), jnp.float32)])
  src3 = src_p.reshape(NW, nchunk, CH)
  dst3 = dst_p.reshape(NW, nchunk, CH)
  w3 = w_p.reshape(NW, nchunk, CH)
  wrep = jnp.broadcast_to(w_p[:, None], (epad, 16)).reshape(NW, nchunk, CH, 16)
  zeros_rows = jnp.zeros((CH, d), jnp.float32)

  pdeg = _make_deg_kernel(npad, nchunk, tslice)(dst3, w3)
  pdegT = pdeg.reshape(NC, npad).T

  rb = 1000
  nb = n // rb
  y = pl.pallas_call(
      _y_body,
      grid=(nb,),
      in_specs=[
          pl.BlockSpec((rb, NC), lambda i: (i, 0)),
          pl.BlockSpec((rb, d), lambda i: (i, 0)),
      ],
      out_specs=pl.BlockSpec((rb, d), lambda i: (i, 0)),
      out_shape=jax.ShapeDtypeStruct((n, d), jnp.float32),
  )(pdegT, x)

  accs = _make_agg_kernel(npad, nchunk, tslice, d)(
      src3, dst3, wrep, y, zeros_rows).reshape(NC, npad, d)

  s = pl.pallas_call(
      _mlp_body,
      grid=(nb,),
      in_specs=[
          pl.BlockSpec((rb, NC), lambda i: (i, 0)),
          pl.BlockSpec((NC, rb, d), lambda i: (0, i, 0)),
          pl.BlockSpec((rb, d), lambda i: (i, 0)),
          pl.BlockSpec((h, d), lambda i: (0, 0)),
          pl.BlockSpec((1, h), lambda i: (0, 0)),
          pl.BlockSpec((mlp, h), lambda i: (0, 0)),
          pl.BlockSpec((1, mlp), lambda i: (0, 0)),
          pl.BlockSpec((k, mlp), lambda i: (0, 0)),
          pl.BlockSpec((1, k), lambda i: (0, 0)),
      ],
      out_specs=pl.BlockSpec((rb, k), lambda i: (i, 0)),
      out_shape=jax.ShapeDtypeStruct((n, k), jnp.float32),
  )(pdegT, accs, y, Wg, bg.reshape(1, h), W1, b1.reshape(1, mlp),
    W2, b2.reshape(1, k))

  return (A, s)


# speedup vs baseline: 15.5754x; 1.1732x over previous
import functools

import jax
import jax.numpy as jnp
from jax import lax
from jax.experimental import pallas as pl
from jax.experimental.pallas import tpu as pltpu
from jax.experimental.pallas import tpu_sc as plsc

NC = 2
NS = 16
NW = NC * NS
CH = 128

def _sc_mesh():
  return plsc.VectorSubcoreMesh(
      core_axis_name="c", subcore_axis_name="s", num_cores=NC, num_subcores=NS
  )


def _make_deg_kernel(npad, nchunk, tslice):
  @functools.partial(
      pl.kernel,
      out_type=jax.ShapeDtypeStruct((NC * npad,), jnp.float32),
      mesh=_sc_mesh(),
      scratch_types=[
          pltpu.VMEM_SHARED((npad,), jnp.float32),
          pltpu.VMEM((nchunk, CH), jnp.int32),
          pltpu.VMEM((nchunk, CH), jnp.float32),
          pltpu.VMEM((tslice,), jnp.float32),
      ],
  )
  def deg_kernel(dst_hbm, w_hbm, out_hbm, deg_sh, didx, wv, zb):
    c = lax.axis_index("c")
    t = lax.axis_index("s")
    wid = c * NS + t

    def zf(k, _):
      zb[pl.ds(k * 16, 16)] = jnp.zeros((16,), jnp.float32)
      return 0

    lax.fori_loop(0, tslice // 16, zf, 0)
    pltpu.sync_copy(zb, deg_sh.at[pl.ds(t * tslice, tslice)])
    pltpu.sync_copy(dst_hbm.at[wid], didx)
    pltpu.sync_copy(w_hbm.at[wid], wv)
    plsc.subcore_barrier()

    def body(i, _):
      pltpu.sync_copy(wv.at[i], deg_sh.at[didx.at[i]], add=True)
      return 0

    lax.fori_loop(0, nchunk, body, 0)
    plsc.subcore_barrier()
    pltpu.sync_copy(deg_sh.at[pl.ds(t * tslice, tslice)], zb)
    pltpu.sync_copy(zb, out_hbm.at[pl.ds(c * npad + t * tslice, tslice)])

  return deg_kernel


def _make_agg_kernel(npad, nchunk, tslice, d):
  @functools.partial(
      pl.kernel,
      out_type=jax.ShapeDtypeStruct((NC * npad, d), jnp.float32),
      mesh=_sc_mesh(),
      scratch_types=[
          pltpu.VMEM_SHARED((npad, d), jnp.float32),
          pltpu.VMEM((CH,), jnp.int32),
          pltpu.VMEM((1, CH), jnp.int32),
          pltpu.VMEM((CH,), jnp.float32),
          pltpu.VMEM((CH, d), jnp.float32),
          pltpu.SemaphoreType.DMA,
      ],
  )
  def agg_kernel(src_hbm, dst_hbm, w_hbm, y_hbm, zeros_hbm, out_hbm,
                 acc_sh, sidx, didx, wrv, rows, gsem):
    c = lax.axis_index("c")
    t = lax.axis_index("s")
    wid = c * NS + t

    pltpu.sync_copy(zeros_hbm, rows)
    nfull = tslice // CH
    rem = tslice % CH

    def zrb(j, _):
      pltpu.sync_copy(rows, acc_sh.at[pl.ds(t * tslice + j * CH, CH)])
      return 0

    lax.fori_loop(0, nfull, zrb, 0)
    if rem:
      pltpu.sync_copy(rows.at[pl.ds(0, rem)],
                      acc_sh.at[pl.ds(t * tslice + nfull * CH, rem)])
    plsc.subcore_barrier()

    def scale(rows_ref, wref):
      def sg(g, _):
        w16 = wref[pl.ds(g * 16, 16)]

        def sl(l, _):
          e = g * 16 + l
          s = w16.at[jnp.full((16,), l, jnp.int32)].get(
              mode="promise_in_bounds")
          for j in range(d // 16):
            rows_ref[e, pl.ds(j * 16, 16)] = (
                rows_ref[e, pl.ds(j * 16, 16)] * s)
          return 0

        lax.fori_loop(0, 16, sl, 0)
        return 0

      lax.fori_loop(0, CH // 16, sg, 0)

    def body(i, _):
      pltpu.sync_copy(src_hbm.at[wid, i], sidx)
      pltpu.async_copy(y_hbm.at[sidx], rows, gsem)
      pltpu.sync_copy(dst_hbm.at[wid, pl.ds(i, 1)], didx)
      pltpu.sync_copy(w_hbm.at[wid, i], wrv)
      pltpu.make_async_copy(y_hbm.at[sidx], rows, gsem).wait()
      scale(rows, wrv)
      pltpu.sync_copy(rows, acc_sh.at[didx.at[0]], add=True)
      return 0

    lax.fori_loop(0, nchunk, body, 0)
    plsc.subcore_barrier()

    def outb(j, _):
      base = t * tslice + j * CH
      pltpu.sync_copy(acc_sh.at[pl.ds(base, CH)], rows)
      pltpu.sync_copy(rows, out_hbm.at[pl.ds(c * npad + base, CH)])
      return 0

    lax.fori_loop(0, nfull, outb, 0)
    if rem:
      base = t * tslice + nfull * CH
      pltpu.sync_copy(acc_sh.at[pl.ds(base, rem)], rows.at[pl.ds(0, rem)])
      pltpu.sync_copy(rows.at[pl.ds(0, rem)],
                      out_hbm.at[pl.ds(c * npad + base, rem)])

  return agg_kernel


def _y_body(pdegT_ref, x_ref, y_ref):
  pd = pdegT_ref[...]
  deg = 1.0 + pd[:, 0:1] + pd[:, 1:2]
  dinv = jnp.where(deg > 0, lax.rsqrt(deg), 0.0)
  y_ref[...] = x_ref[...] * dinv


def _mlp_body(pdegT_ref, acc_ref, y_ref, wg_ref, bg_ref, w1_ref, b1_ref,
              w2_ref, b2_ref, out_ref):
  pd = pdegT_ref[...]
  deg = 1.0 + pd[:, 0:1] + pd[:, 1:2]
  dinv = jnp.where(deg > 0, lax.rsqrt(deg), 0.0)
  acc = acc_ref[...]
  agg = (acc[0] + acc[1] + y_ref[...]) * dinv
  dn = (((1,), (1,)), ((), ()))
  h = lax.dot_general(agg, wg_ref[...], dn,
                      preferred_element_type=jnp.float32) + bg_ref[...]
  h = h * (1.0 / (1.0 + jnp.exp(-h)))
  z = lax.dot_general(h, w1_ref[...], dn,
                      preferred_element_type=jnp.float32) + b1_ref[...]
  z = jnp.where(z > 0, z, jnp.exp(z) - 1.0)
  hl = lax.dot_general(z, w2_ref[...], dn,
                       preferred_element_type=jnp.float32) + b2_ref[...]
  m = jnp.max(hl, axis=1, keepdims=True)
  ex = jnp.exp(hl - m)
  out_ref[...] = ex / jnp.sum(ex, axis=1, keepdims=True)


def kernel(x, edge_index, edge_attr, A, Wg, bg, W1, b1, W2, b2):
  n, d = x.shape
  e = edge_index.shape[1]
  h = Wg.shape[0]
  mlp = W1.shape[0]
  k = W2.shape[0]

  tslice = -(-n // (NS * 8)) * 8
  npad = tslice * NS
  epw = -(-e // (NW * CH)) * CH
  nchunk = epw // CH
  epad = epw * NW

  src = edge_index[0]
  dst = edge_index[1]
  ne_pad = epad - e
  pad_idx = jnp.arange(ne_pad, dtype=jnp.int32) % n
  src_p = jnp.concatenate([src, pad_idx])
  dst_p = jnp.concatenate([dst, pad_idx])
  w_p = jnp.concatenate([edge_attr, jnp.zeros((ne_pad,), jnp.float32)])
  src3 = src_p.reshape(NW, nchunk, CH)
  dst3 = dst_p.reshape(NW, nchunk, CH)
  w3 = w_p.reshape(NW, nchunk, CH)
  zeros_rows = jnp.zeros((CH, d), jnp.float32)

  pdeg = _make_deg_kernel(npad, nchunk, tslice)(dst3, w3)
  pdegT = pdeg.reshape(NC, npad).T

  rb = 1000
  nb = n // rb
  y = pl.pallas_call(
      _y_body,
      grid=(nb,),
      in_specs=[
          pl.BlockSpec((rb, NC), lambda i: (i, 0)),
          pl.BlockSpec((rb, d), lambda i: (i, 0)),
      ],
      out_specs=pl.BlockSpec((rb, d), lambda i: (i, 0)),
      out_shape=jax.ShapeDtypeStruct((n, d), jnp.float32),
  )(pdegT, x)

  accs = _make_agg_kernel(npad, nchunk, tslice, d)(
      src3, dst3, w3, y, zeros_rows).reshape(NC, npad, d)

  s = pl.pallas_call(
      _mlp_body,
      grid=(nb,),
      in_specs=[
          pl.BlockSpec((rb, NC), lambda i: (i, 0)),
          pl.BlockSpec((NC, rb, d), lambda i: (0, i, 0)),
          pl.BlockSpec((rb, d), lambda i: (i, 0)),
          pl.BlockSpec((h, d), lambda i: (0, 0)),
          pl.BlockSpec((1, h), lambda i: (0, 0)),
          pl.BlockSpec((mlp, h), lambda i: (0, 0)),
          pl.BlockSpec((1, mlp), lambda i: (0, 0)),
          pl.BlockSpec((k, mlp), lambda i: (0, 0)),
          pl.BlockSpec((1, k), lambda i: (0, 0)),
      ],
      out_specs=pl.BlockSpec((rb, k), lambda i: (i, 0)),
      out_shape=jax.ShapeDtypeStruct((n, k), jnp.float32),
  )(pdegT, accs, y, Wg, bg.reshape(1, h), W1, b1.reshape(1, mlp),
    W2, b2.reshape(1, k))

  return (A, s)
